# baseline (speedup 1.0000x reference)
import jax
import jax.numpy as jnp
from jax import lax
from jax.experimental import pallas as pl
from jax.experimental.pallas import tpu as pltpu
from jax.experimental.pallas import tpu_sc as plsc

NDIM = 256
HALF = 128
N_NODES = 10000
NPAD = 10240
E_EDGES = 160000
NC, NS = 2, 16
CH = 64
SBLK = 16
NBLK = 10
EPAD = NS * NBLK * SBLK * CH
TRASH = N_NODES
ROWS_PT = NPAD // NS
NODE_CH = ROWS_PT // CH

_SC_MESH = plsc.VectorSubcoreMesh(core_axis_name="c", subcore_axis_name="s")


def _sc_prep_body(atts_hbm, emb_hbm, h_hbm, atts_v, rows_v):
    c = lax.axis_index("c")
    s = lax.axis_index("s")
    pltpu.sync_copy(atts_hbm.at[s], atts_v)
    for j in range(NODE_CH):
        pltpu.sync_copy(emb_hbm.at[c].at[atts_v.at[j]], rows_v)
        pltpu.sync_copy(rows_v,
                        h_hbm.at[c, pl.ds(s * ROWS_PT + j * CH, CH)])


_sc_prep = pl.kernel(
    _sc_prep_body,
    out_type=jax.ShapeDtypeStruct((NC, NPAD, HALF), jnp.float32),
    mesh=_SC_MESH,
    scratch_types=[
        pltpu.VMEM((NODE_CH, CH), jnp.int32),
        pltpu.VMEM((CH, HALF), jnp.float32),
    ],
)


def _sc_aggr_body(uf_hbm, ur_hbm, eif_hbm, eir_hbm, zeros_hbm,
                  gin_hbm, gout_hbm,
                  acc, gidx, sidx, rows, semi, semj,
                  sg0, sg1, sg2, sg3, ss0, ss1, ss2, ss3):
    c = lax.axis_index("c")
    s = lax.axis_index("s")
    SG = (sg0, sg1, sg2, sg3)
    SS = (ss0, ss1, ss2, ss3)
    NCH = NBLK * SBLK

    def phase(tab, gsrc, gdst, out_hbm, g):
        def gs(islot, kl, slot):
            pltpu.async_copy(tab.at[gidx.at[islot, kl]], rows.at[slot],
                             SG[slot])

        def gw(slot):
            pltpu.make_async_copy(tab.at[gidx.at[0, 0]], rows.at[slot],
                                  SG[slot]).wait()

        def sstart(islot, kl, slot):
            pltpu.async_copy(rows.at[slot], acc.at[sidx.at[islot, kl]],
                             SS[slot], add=True)

        def sw(slot):
            pltpu.make_async_copy(rows.at[slot], acc.at[sidx.at[0, 0]],
                                  SS[slot]).wait()

        def chunk_static(b, kl, do_sw=True):
            k = SBLK * b + kl
            r = kl % 4
            if do_sw:
                sw((r + 2) % 4)
            if k + 2 < NCH:
                kl2 = kl + 2
                if kl2 < SBLK:
                    gs(b % 3, kl2, (r + 2) % 4)
                else:
                    gs((b + 1) % 3, kl2 - SBLK, (r + 2) % 4)
            gw(r)
            sstart(b % 3, kl, r)

        pltpu.sync_copy(zeros_hbm, acc.at[pl.ds(s * ROWS_PT, ROWS_PT)])
        plsc.subcore_barrier()
        pltpu.sync_copy(gsrc.at[s, 0], gidx.at[0])
        pltpu.sync_copy(gdst.at[s, 0], sidx.at[0])
        pltpu.async_copy(gsrc.at[s, 1], gidx.at[1], semi)
        pltpu.async_copy(gdst.at[s, 1], sidx.at[1], semj)
        gs(0, 0, 0)
        gs(0, 1, 1)

        for b in range(NBLK):
            ib = b % 3
            if b == 0:
                chunk_static(0, 0, do_sw=False)
                chunk_static(0, 1, do_sw=False)
                chunk_static(0, 2)
                chunk_static(0, 3)
                lo = 1
            else:
                lo = 0

            def body(jj, carry, ib=ib):
                for r in range(4):
                    kl = 4 * jj + r
                    sw((r + 2) % 4)
                    gs(ib, kl + 2, (r + 2) % 4)
                    gw(r)
                    sstart(ib, kl, r)
                return carry

            lax.fori_loop(lo, (SBLK - 4) // 4, body, 0)
            if b + 1 < NBLK:
                pltpu.make_async_copy(gsrc.at[s, b + 1],
                                      gidx.at[(b + 1) % 3], semi).wait()
                pltpu.make_async_copy(gdst.at[s, b + 1],
                                      sidx.at[(b + 1) % 3], semj).wait()
            if b + 2 < NBLK:
                pltpu.async_copy(gsrc.at[s, b + 2], gidx.at[(b + 2) % 3],
                                 semi)
                pltpu.async_copy(gdst.at[s, b + 2], sidx.at[(b + 2) % 3],
                                 semj)
            for kl in range(SBLK - 4, SBLK):
                chunk_static(b, kl)

        sw(2)
        sw(3)
        plsc.subcore_barrier()
        pltpu.sync_copy(acc.at[pl.ds(s * ROWS_PT, ROWS_PT)],
                        out_hbm.at[g, pl.ds(s * ROWS_PT, ROWS_PT)])
        plsc.subcore_barrier()

    for tab4, gsrc, gdst, out in ((uf_hbm, eif_hbm, eir_hbm, gin_hbm),
                                  (ur_hbm, eir_hbm, eif_hbm, gout_hbm)):
        for sub in range(2):
            g = 2 * sub + c
            phase(tab4.at[g], gsrc, gdst, out, g)


_sc_aggr = pl.kernel(
    _sc_aggr_body,
    out_type=(
        jax.ShapeDtypeStruct((4, NPAD, HALF), jnp.float32),
        jax.ShapeDtypeStruct((4, NPAD, HALF), jnp.float32),
    ),
    mesh=_SC_MESH,
    scratch_types=[
        pltpu.VMEM_SHARED((NPAD, HALF), jnp.float32),
        pltpu.VMEM((3, SBLK, CH), jnp.int32),
        pltpu.VMEM((3, SBLK, CH), jnp.int32),
        pltpu.VMEM((4, CH, HALF), jnp.float32),
        pltpu.SemaphoreType.DMA,
        pltpu.SemaphoreType.DMA,
        pltpu.SemaphoreType.DMA,
        pltpu.SemaphoreType.DMA,
        pltpu.SemaphoreType.DMA,
        pltpu.SemaphoreType.DMA,
        pltpu.SemaphoreType.DMA,
        pltpu.SemaphoreType.DMA,
        pltpu.SemaphoreType.DMA,
        pltpu.SemaphoreType.DMA,
    ],
)


def _sc_deg_body(eid_hbm, zeros_hbm, deg_hbm, acc, didx, ones_v, semi, sems):
    c = lax.axis_index("c")
    s = lax.axis_index("s")

    def fill_ones(i, carry):
        for j in range(HALF // 16):
            ones_v[i, pl.ds(j * 16, 16)] = jnp.ones((16,), jnp.float32)
        return carry

    lax.fori_loop(0, CH, fill_ones, 0)
    pltpu.sync_copy(zeros_hbm, acc.at[pl.ds(s * ROWS_PT, ROWS_PT)])
    plsc.subcore_barrier()

    tabi = eid_hbm.at[c]
    pltpu.sync_copy(tabi.at[s, 0], didx.at[0])
    for b in range(NBLK):
        cur = b % 2
        nxt = (b + 1) % 2
        if b + 1 < NBLK:
            pltpu.async_copy(tabi.at[s, b + 1], didx.at[nxt], semi)

        pltpu.async_copy(ones_v, acc.at[didx.at[cur, 0]], sems, add=True)
        pltpu.async_copy(ones_v, acc.at[didx.at[cur, 1]], sems, add=True)

        def fire(jj, carry):
            pltpu.make_async_copy(ones_v, acc.at[didx.at[cur, 0]],
                                  sems).wait()
            pltpu.async_copy(ones_v, acc.at[didx.at[cur, jj + 2]], sems,
                             add=True)
            return carry

        lax.fori_loop(0, SBLK - 2, fire, 0)
        pltpu.make_async_copy(ones_v, acc.at[didx.at[cur, 0]], sems).wait()
        pltpu.make_async_copy(ones_v, acc.at[didx.at[cur, 0]], sems).wait()
        if b + 1 < NBLK:
            pltpu.make_async_copy(tabi.at[s, b + 1], didx.at[nxt],
                                  semi).wait()

    plsc.subcore_barrier()
    pltpu.sync_copy(acc.at[pl.ds(s * ROWS_PT, ROWS_PT)],
                    deg_hbm.at[c, pl.ds(s * ROWS_PT, ROWS_PT)])


_sc_deg = pl.kernel(
    _sc_deg_body,
    out_type=jax.ShapeDtypeStruct((NC, NPAD, HALF), jnp.float32),
    mesh=_SC_MESH,
    scratch_types=[
        pltpu.VMEM_SHARED((NPAD, HALF), jnp.float32),
        pltpu.VMEM((2, SBLK, CH), jnp.int32),
        pltpu.VMEM((CH, HALF), jnp.float32),
        pltpu.SemaphoreType.DMA,
        pltpu.SemaphoreType.DMA,
    ],
)


_BLK = 256
_GRID = NPAD // _BLK


def _tc_u_body(h_ref, wjf_ref, wjr_ref, uf_ref, ur_ref):
    f32 = jnp.float32
    X = jnp.concatenate([h_ref[0], h_ref[1]], axis=1)
    uf = jnp.dot(X, wjf_ref[...], preferred_element_type=f32)
    ur = jnp.dot(X, wjr_ref[...], preferred_element_type=f32)
    for g in range(4):
        uf_ref[g] = uf[:, g * HALF:(g + 1) * HALF]
        ur_ref[g] = ur[:, g * HALF:(g + 1) * HALF]


def _tc_u(h, wjf, wjr):
    grp4 = pl.BlockSpec((4, _BLK, HALF), lambda i: (0, i, 0))
    return pl.pallas_call(
        _tc_u_body,
        grid=(_GRID,),
        in_specs=[
            pl.BlockSpec((2, _BLK, HALF), lambda i: (0, i, 0)),
            pl.BlockSpec((NDIM, 2 * NDIM), lambda i: (0, 0)),
            pl.BlockSpec((NDIM, 2 * NDIM), lambda i: (0, 0)),
        ],
        out_specs=[grp4, grp4],
        out_shape=[
            jax.ShapeDtypeStruct((4, NPAD, HALF), jnp.float32),
            jax.ShapeDtypeStruct((4, NPAD, HALF), jnp.float32),
        ],
    )(h, wjf, wjr)


def _tc_dense_body(h_ref, g_ref, go_ref, din_ref, dout_ref,
                   wv_ref, wih_ref, whh_ref,
                   bf_ref, br_ref, bih_ref, bhh_ref, out_ref):
    f32 = jnp.float32
    X = jnp.concatenate([h_ref[0], h_ref[1]], axis=1)
    S = (jnp.concatenate([g_ref[0], g_ref[1], g_ref[2], g_ref[3]], axis=1)
         + jnp.concatenate([go_ref[0], go_ref[1], go_ref[2], go_ref[3]],
                           axis=1))
    V = jnp.dot(X, wv_ref[...], preferred_element_type=f32)
    din = din_ref[0][:, 0:1]
    dout = dout_ref[0][:, 0:1]
    aggr = (S + din * (V[:, :2 * NDIM] + bf_ref[...])
            + dout * (V[:, 2 * NDIM:] + br_ref[...]))
    gi = jnp.dot(aggr, wih_ref[...],
                 preferred_element_type=f32) + bih_ref[...]
    gh = jnp.dot(X, whh_ref[...],
                 preferred_element_type=f32) + bhh_ref[...]
    r = jax.nn.sigmoid(gi[:, :NDIM] + gh[:, :NDIM])
    z = jax.nn.sigmoid(gi[:, NDIM:2 * NDIM] + gh[:, NDIM:2 * NDIM])
    n = jnp.tanh(gi[:, 2 * NDIM:] + r * gh[:, 2 * NDIM:])
    hn = (1.0 - z) * n + z * X
    out_ref[0] = hn[:, :HALF]
    out_ref[1] = hn[:, HALF:]


def _tc_dense(h, gin, gout, din, dout, wv, wih, whh, bf, br, bih, bhh):
    grp4 = pl.BlockSpec((4, _BLK, HALF), lambda i: (0, i, 0))
    split3 = pl.BlockSpec((2, _BLK, HALF), lambda i: (0, i, 0))
    d3 = pl.BlockSpec((1, _BLK, HALF), lambda i: (0, i, 0))
    full2 = lambda shape: pl.BlockSpec(shape, lambda i: (0, 0))
    return pl.pallas_call(
        _tc_dense_body,
        grid=(_GRID,),
        in_specs=[
            split3,
            grp4, grp4,
            d3, pl.BlockSpec((1, _BLK, HALF), lambda i: (1, i, 0)),
            full2((NDIM, 4 * NDIM)),
            full2((2 * NDIM, 3 * NDIM)),
            full2((NDIM, 3 * NDIM)),
            full2((1, 2 * NDIM)),
            full2((1, 2 * NDIM)),
            full2((1, 3 * NDIM)),
            full2((1, 3 * NDIM)),
        ],
        out_specs=pl.BlockSpec((2, _BLK, HALF), lambda i: (0, i, 0)),
        out_shape=jax.ShapeDtypeStruct((NC, NPAD, HALF), jnp.float32),
    )(h, gin, gout, din, dout, wv, wih, whh, bf, br, bih, bhh)


def kernel(edge_index, node_atts, emb, msg_W, msg_b, msg_rev_W, msg_rev_b,
           gru_Wih, gru_bih, gru_Whh, gru_bhh):
    pad = jnp.full((EPAD - E_EDGES,), TRASH, jnp.int32)
    eif = jnp.concatenate([edge_index[0], pad]).reshape(NS, NBLK, SBLK, CH)
    eir = jnp.concatenate([edge_index[1], pad]).reshape(NS, NBLK, SBLK, CH)
    atts = jnp.pad(node_atts, (0, NPAD - N_NODES)).reshape(NS, NODE_CH, CH)
    emb2 = jnp.stack([emb[:, :HALF], emb[:, HALF:]])
    zeros = jnp.zeros((ROWS_PT, HALF), jnp.float32)
    eid = jnp.stack([eir, eif])

    h = _sc_prep(atts, emb2)
    h, eid = lax.optimization_barrier((h, eid))
    deg = _sc_deg(eid, zeros)

    num_layers = msg_W.shape[0]
    for l in range(num_layers):
        uf, ur = _tc_u(h, msg_W[l][:, :NDIM].T, msg_rev_W[l][:, :NDIM].T)
        if l == 0:
            uf, ur, deg = lax.optimization_barrier((uf, ur, deg))
        gin, gout = _sc_aggr(uf, ur, eif, eir, zeros)
        wv = jnp.concatenate([msg_W[l][:, NDIM:].T,
                              msg_rev_W[l][:, NDIM:].T], axis=1)
        h = _tc_dense(
            h, gin, gout, deg, deg,
            wv, gru_Wih[l].T, gru_Whh[l].T,
            msg_b[l][None, :], msg_rev_b[l][None, :],
            gru_bih[l][None, :], gru_bhh[l][None, :],
        )

    return jnp.concatenate([h[0], h[1]], axis=1)[:N_NODES]


# speedup vs baseline: 1.0818x; 1.0818x over previous
import jax
import jax.numpy as jnp
from jax import lax
from jax.experimental import pallas as pl
from jax.experimental.pallas import tpu as pltpu
from jax.experimental.pallas import tpu_sc as plsc

NDIM = 256
HALF = 128
N_NODES = 10000
NPAD = 10240
E_EDGES = 160000
NC, NS = 2, 16
CH = 64
SBLK = 16
NBLK = 10
EPAD = NS * NBLK * SBLK * CH
TRASH = N_NODES
ROWS_PT = NPAD // NS
NODE_CH = ROWS_PT // CH

_SC_MESH = plsc.VectorSubcoreMesh(core_axis_name="c", subcore_axis_name="s")


def _sc_prep_body(atts_hbm, emb_hbm, h_hbm, atts_v, rows_v):
    c = lax.axis_index("c")
    s = lax.axis_index("s")
    pltpu.sync_copy(atts_hbm.at[s], atts_v)
    for j in range(NODE_CH):
        pltpu.sync_copy(emb_hbm.at[c].at[atts_v.at[j]], rows_v)
        pltpu.sync_copy(rows_v,
                        h_hbm.at[c, pl.ds(s * ROWS_PT + j * CH, CH)])


_sc_prep = pl.kernel(
    _sc_prep_body,
    out_type=jax.ShapeDtypeStruct((NC, NPAD, HALF), jnp.float32),
    mesh=_SC_MESH,
    scratch_types=[
        pltpu.VMEM((NODE_CH, CH), jnp.int32),
        pltpu.VMEM((CH, HALF), jnp.float32),
    ],
)


def _sc_aggr_body(uf_hbm, ur_hbm, eif_hbm, eir_hbm, zeros_hbm,
                  gin_hbm, gout_hbm,
                  acc, gidx, sidx, rows, semi, semj,
                  sg0, sg1, sg2, sg3, ss0, ss1, ss2, ss3):
    c = lax.axis_index("c")
    s = lax.axis_index("s")
    SG = (sg0, sg1, sg2, sg3)
    SS = (ss0, ss1, ss2, ss3)
    NCH = NBLK * SBLK

    def phase(tab, gsrc, gdst, out_hbm, g):
        def gs(islot, kl, slot):
            pltpu.async_copy(tab.at[gidx.at[islot, kl]], rows.at[slot],
                             SG[slot])

        def gw(slot):
            pltpu.make_async_copy(tab.at[gidx.at[0, 0]], rows.at[slot],
                                  SG[slot]).wait()

        def sstart(islot, kl, slot):
            pltpu.async_copy(rows.at[slot], acc.at[sidx.at[islot, kl]],
                             SS[slot], add=True)

        def sw(slot):
            pltpu.make_async_copy(rows.at[slot], acc.at[sidx.at[0, 0]],
                                  SS[slot]).wait()

        def chunk_static(b, kl, do_sw=True):
            k = SBLK * b + kl
            r = kl % 4
            if do_sw:
                sw((r + 2) % 4)
            if k + 2 < NCH:
                kl2 = kl + 2
                if kl2 < SBLK:
                    gs(b % 3, kl2, (r + 2) % 4)
                else:
                    gs((b + 1) % 3, kl2 - SBLK, (r + 2) % 4)
            gw(r)
            sstart(b % 3, kl, r)

        pltpu.sync_copy(zeros_hbm, acc.at[pl.ds(s * ROWS_PT, ROWS_PT)])
        plsc.subcore_barrier()
        pltpu.sync_copy(gsrc.at[s, 0], gidx.at[0])
        pltpu.sync_copy(gdst.at[s, 0], sidx.at[0])
        pltpu.async_copy(gsrc.at[s, 1], gidx.at[1], semi)
        pltpu.async_copy(gdst.at[s, 1], sidx.at[1], semj)
        gs(0, 0, 0)
        gs(0, 1, 1)

        for b in range(NBLK):
            ib = b % 3
            if b == 0:
                chunk_static(0, 0, do_sw=False)
                chunk_static(0, 1, do_sw=False)
                chunk_static(0, 2)
                chunk_static(0, 3)
                lo = 1
            else:
                lo = 0

            def body(jj, carry, ib=ib):
                for r in range(4):
                    kl = 4 * jj + r
                    sw((r + 2) % 4)
                    gs(ib, kl + 2, (r + 2) % 4)
                    gw(r)
                    sstart(ib, kl, r)
                return carry

            lax.fori_loop(lo, (SBLK - 4) // 4, body, 0)
            if b + 1 < NBLK:
                pltpu.make_async_copy(gsrc.at[s, b + 1],
                                      gidx.at[(b + 1) % 3], semi).wait()
                pltpu.make_async_copy(gdst.at[s, b + 1],
                                      sidx.at[(b + 1) % 3], semj).wait()
            if b + 2 < NBLK:
                pltpu.async_copy(gsrc.at[s, b + 2], gidx.at[(b + 2) % 3],
                                 semi)
                pltpu.async_copy(gdst.at[s, b + 2], sidx.at[(b + 2) % 3],
                                 semj)
            for kl in range(SBLK - 4, SBLK):
                chunk_static(b, kl)

        sw(2)
        sw(3)
        plsc.subcore_barrier()
        pltpu.sync_copy(acc.at[pl.ds(s * ROWS_PT, ROWS_PT)],
                        out_hbm.at[g, pl.ds(s * ROWS_PT, ROWS_PT)])
        plsc.subcore_barrier()

    for tab4, gsrc, gdst, out in ((uf_hbm, eif_hbm, eir_hbm, gin_hbm),
                                  (ur_hbm, eir_hbm, eif_hbm, gout_hbm)):
        for sub in range(2):
            g = 2 * sub + c
            phase(tab4.at[g], gsrc, gdst, out, g)


_sc_aggr = pl.kernel(
    _sc_aggr_body,
    out_type=(
        jax.ShapeDtypeStruct((4, NPAD, HALF), jnp.float32),
        jax.ShapeDtypeStruct((4, NPAD, HALF), jnp.float32),
    ),
    mesh=_SC_MESH,
    scratch_types=[
        pltpu.VMEM_SHARED((NPAD, HALF), jnp.float32),
        pltpu.VMEM((3, SBLK, CH), jnp.int32),
        pltpu.VMEM((3, SBLK, CH), jnp.int32),
        pltpu.VMEM((4, CH, HALF), jnp.float32),
        pltpu.SemaphoreType.DMA,
        pltpu.SemaphoreType.DMA,
        pltpu.SemaphoreType.DMA,
        pltpu.SemaphoreType.DMA,
        pltpu.SemaphoreType.DMA,
        pltpu.SemaphoreType.DMA,
        pltpu.SemaphoreType.DMA,
        pltpu.SemaphoreType.DMA,
        pltpu.SemaphoreType.DMA,
        pltpu.SemaphoreType.DMA,
    ],
)


def _sc_deg_body(eid_hbm, zeros_hbm, deg_hbm, acc, didx, ones_v, semi, sems):
    c = lax.axis_index("c")
    s = lax.axis_index("s")

    def fill_ones(i, carry):
        for j in range(HALF // 16):
            ones_v[i, pl.ds(j * 16, 16)] = jnp.ones((16,), jnp.float32)
        return carry

    lax.fori_loop(0, CH, fill_ones, 0)
    pltpu.sync_copy(zeros_hbm, acc.at[pl.ds(s * ROWS_PT, ROWS_PT)])
    plsc.subcore_barrier()

    tabi = eid_hbm.at[c]
    pltpu.sync_copy(tabi.at[s, 0], didx.at[0])
    for b in range(NBLK):
        cur = b % 2
        nxt = (b + 1) % 2
        if b + 1 < NBLK:
            pltpu.async_copy(tabi.at[s, b + 1], didx.at[nxt], semi)

        pltpu.async_copy(ones_v, acc.at[didx.at[cur, 0]], sems, add=True)
        pltpu.async_copy(ones_v, acc.at[didx.at[cur, 1]], sems, add=True)

        def fire(jj, carry):
            pltpu.make_async_copy(ones_v, acc.at[didx.at[cur, 0]],
                                  sems).wait()
            pltpu.async_copy(ones_v, acc.at[didx.at[cur, jj + 2]], sems,
                             add=True)
            return carry

        lax.fori_loop(0, SBLK - 2, fire, 0)
        pltpu.make_async_copy(ones_v, acc.at[didx.at[cur, 0]], sems).wait()
        pltpu.make_async_copy(ones_v, acc.at[didx.at[cur, 0]], sems).wait()
        if b + 1 < NBLK:
            pltpu.make_async_copy(tabi.at[s, b + 1], didx.at[nxt],
                                  semi).wait()

    plsc.subcore_barrier()
    pltpu.sync_copy(acc.at[pl.ds(s * ROWS_PT, ROWS_PT)],
                    deg_hbm.at[c, pl.ds(s * ROWS_PT, ROWS_PT)])


_sc_deg = pl.kernel(
    _sc_deg_body,
    out_type=jax.ShapeDtypeStruct((NC, NPAD, HALF), jnp.float32),
    mesh=_SC_MESH,
    scratch_types=[
        pltpu.VMEM_SHARED((NPAD, HALF), jnp.float32),
        pltpu.VMEM((2, SBLK, CH), jnp.int32),
        pltpu.VMEM((CH, HALF), jnp.float32),
        pltpu.SemaphoreType.DMA,
        pltpu.SemaphoreType.DMA,
    ],
)


_BLK = 256
_GRID = NPAD // _BLK


def _tc_u_body(h_ref, wjf_ref, wjr_ref, uf_ref, ur_ref):
    f32 = jnp.float32
    X = jnp.concatenate([h_ref[0], h_ref[1]], axis=1)
    uf = jnp.dot(X, wjf_ref[...], preferred_element_type=f32)
    ur = jnp.dot(X, wjr_ref[...], preferred_element_type=f32)
    for g in range(4):
        uf_ref[g] = uf[:, g * HALF:(g + 1) * HALF]
        ur_ref[g] = ur[:, g * HALF:(g + 1) * HALF]


def _tc_u(h, wjf, wjr):
    grp4 = pl.BlockSpec((4, _BLK, HALF), lambda i: (0, i, 0))
    return pl.pallas_call(
        _tc_u_body,
        grid=(_GRID,),
        in_specs=[
            pl.BlockSpec((2, _BLK, HALF), lambda i: (0, i, 0)),
            pl.BlockSpec((NDIM, 2 * NDIM), lambda i: (0, 0)),
            pl.BlockSpec((NDIM, 2 * NDIM), lambda i: (0, 0)),
        ],
        out_specs=[grp4, grp4],
        out_shape=[
            jax.ShapeDtypeStruct((4, NPAD, HALF), jnp.float32),
            jax.ShapeDtypeStruct((4, NPAD, HALF), jnp.float32),
        ],
    )(h, wjf, wjr)


def _make_tc_dense(emit_u):
    def body(*refs):
        if emit_u:
            (h_ref, g_ref, go_ref, din_ref, dout_ref, wv_ref, wih_ref,
             whh_ref, bf_ref, br_ref, bih_ref, bhh_ref, wjf_ref, wjr_ref,
             out_ref, uf_ref, ur_ref) = refs
        else:
            (h_ref, g_ref, go_ref, din_ref, dout_ref, wv_ref, wih_ref,
             whh_ref, bf_ref, br_ref, bih_ref, bhh_ref, out_ref) = refs
        f32 = jnp.float32
        X = jnp.concatenate([h_ref[0], h_ref[1]], axis=1)
        S = (jnp.concatenate([g_ref[0], g_ref[1], g_ref[2], g_ref[3]],
                             axis=1)
             + jnp.concatenate([go_ref[0], go_ref[1], go_ref[2],
                                go_ref[3]], axis=1))
        V = jnp.dot(X, wv_ref[...], preferred_element_type=f32)
        din = din_ref[0][:, 0:1]
        dout = dout_ref[0][:, 0:1]
        aggr = (S + din * (V[:, :2 * NDIM] + bf_ref[...])
                + dout * (V[:, 2 * NDIM:] + br_ref[...]))
        gi = jnp.dot(aggr, wih_ref[...],
                     preferred_element_type=f32) + bih_ref[...]
        gh = jnp.dot(X, whh_ref[...],
                     preferred_element_type=f32) + bhh_ref[...]
        r = jax.nn.sigmoid(gi[:, :NDIM] + gh[:, :NDIM])
        z = jax.nn.sigmoid(gi[:, NDIM:2 * NDIM] + gh[:, NDIM:2 * NDIM])
        n = jnp.tanh(gi[:, 2 * NDIM:] + r * gh[:, 2 * NDIM:])
        hn = (1.0 - z) * n + z * X
        out_ref[0] = hn[:, :HALF]
        out_ref[1] = hn[:, HALF:]
        if emit_u:
            uf = jnp.dot(hn, wjf_ref[...], preferred_element_type=f32)
            ur = jnp.dot(hn, wjr_ref[...], preferred_element_type=f32)
            for g in range(4):
                uf_ref[g] = uf[:, g * HALF:(g + 1) * HALF]
                ur_ref[g] = ur[:, g * HALF:(g + 1) * HALF]

    grp4 = pl.BlockSpec((4, _BLK, HALF), lambda i: (0, i, 0))
    split3 = pl.BlockSpec((2, _BLK, HALF), lambda i: (0, i, 0))
    d3 = pl.BlockSpec((1, _BLK, HALF), lambda i: (0, i, 0))
    full2 = lambda shape: pl.BlockSpec(shape, lambda i: (0, 0))
    in_specs = [
        split3,
        grp4, grp4,
        d3, pl.BlockSpec((1, _BLK, HALF), lambda i: (1, i, 0)),
        full2((NDIM, 4 * NDIM)),
        full2((2 * NDIM, 3 * NDIM)),
        full2((NDIM, 3 * NDIM)),
        full2((1, 2 * NDIM)),
        full2((1, 2 * NDIM)),
        full2((1, 3 * NDIM)),
        full2((1, 3 * NDIM)),
    ]
    out_specs = pl.BlockSpec((2, _BLK, HALF), lambda i: (0, i, 0))
    out_shape = jax.ShapeDtypeStruct((NC, NPAD, HALF), jnp.float32)
    if emit_u:
        in_specs = in_specs + [full2((NDIM, 2 * NDIM))] * 2
        out_specs = [out_specs, grp4, grp4]
        out_shape = [out_shape,
                     jax.ShapeDtypeStruct((4, NPAD, HALF), jnp.float32),
                     jax.ShapeDtypeStruct((4, NPAD, HALF), jnp.float32)]
    return pl.pallas_call(body, grid=(_GRID,), in_specs=in_specs,
                          out_specs=out_specs, out_shape=out_shape)


_tc_dense = _make_tc_dense(False)
_tc_dense_u = _make_tc_dense(True)


def kernel(edge_index, node_atts, emb, msg_W, msg_b, msg_rev_W, msg_rev_b,
           gru_Wih, gru_bih, gru_Whh, gru_bhh):
    pad = jnp.full((EPAD - E_EDGES,), TRASH, jnp.int32)
    eif = jnp.concatenate([edge_index[0], pad]).reshape(NS, NBLK, SBLK, CH)
    eir = jnp.concatenate([edge_index[1], pad]).reshape(NS, NBLK, SBLK, CH)
    atts = jnp.pad(node_atts, (0, NPAD - N_NODES)).reshape(NS, NODE_CH, CH)
    emb2 = jnp.stack([emb[:, :HALF], emb[:, HALF:]])
    zeros = jnp.zeros((ROWS_PT, HALF), jnp.float32)
    eid = jnp.stack([eir, eif])

    h = _sc_prep(atts, emb2)
    h, eid = lax.optimization_barrier((h, eid))
    deg = _sc_deg(eid, zeros)

    num_layers = msg_W.shape[0]
    uf, ur = _tc_u(h, msg_W[0][:, :NDIM].T, msg_rev_W[0][:, :NDIM].T)
    uf, ur, deg = lax.optimization_barrier((uf, ur, deg))
    for l in range(num_layers):
        gin, gout = _sc_aggr(uf, ur, eif, eir, zeros)
        wv = jnp.concatenate([msg_W[l][:, NDIM:].T,
                              msg_rev_W[l][:, NDIM:].T], axis=1)
        args = (h, gin, gout, deg, deg,
                wv, gru_Wih[l].T, gru_Whh[l].T,
                msg_b[l][None, :], msg_rev_b[l][None, :],
                gru_bih[l][None, :], gru_bhh[l][None, :])
        if l + 1 < num_layers:
            h, uf, ur = _tc_dense_u(*args, msg_W[l + 1][:, :NDIM].T,
                                    msg_rev_W[l + 1][:, :NDIM].T)
        else:
            h = _tc_dense(*args)

    return jnp.concatenate([h[0], h[1]], axis=1)[:N_NODES]
